# baseline (speedup 1.0000x reference)
import jax
import jax.numpy as jnp
from jax.experimental import pallas as pl
from jax.experimental.pallas import tpu as pltpu

H = 64; W = 64; B = 4; C = 128; KQ = 132; KLOC = 7; NLONG = 9; D = 64; ITERS = 16
N = H * W
KTOT = KLOC * KLOC + NLONG


def _kq_body(x_ref, wk_ref, bk_ref, wq_ref, bq_ref, ks_ref, qs_ref):
    x = x_ref[0]
    ks_ref[0] = jnp.dot(x, wk_ref[...], preferred_element_type=jnp.float32,
                        precision=jax.lax.Precision.HIGHEST) + bk_ref[...]
    qs_ref[0] = jnp.dot(x, wq_ref[...], preferred_element_type=jnp.float32,
                        precision=jax.lax.Precision.HIGHEST) + bq_ref[...]


def _compute_kq(x, Wk, bk, Wq, bq):
    return pl.pallas_call(
        _kq_body,
        grid=(B,),
        in_specs=[
            pl.BlockSpec((1, N, C), lambda b: (b, 0, 0)),
            pl.BlockSpec((C, KQ), lambda b: (0, 0)),
            pl.BlockSpec((KQ,), lambda b: (0,)),
            pl.BlockSpec((C, KQ), lambda b: (0, 0)),
            pl.BlockSpec((KQ,), lambda b: (0,)),
        ],
        out_specs=[
            pl.BlockSpec((1, N, KQ), lambda b: (b, 0, 0)),
            pl.BlockSpec((1, N, KQ), lambda b: (b, 0, 0)),
        ],
        out_shape=[
            jax.ShapeDtypeStruct((B, N, KQ), jnp.float32),
            jax.ShapeDtypeStruct((B, N, KQ), jnp.float32),
        ],
    )(x, Wk, bk, Wq, bq)


def kernel(x, v_inds, h0, Wk, bk, Wq, bq):
    ks, qs = _compute_kq(x, Wk, bk, Wq, bq)
    qs_g = jax.vmap(lambda q, idx: q[idx])(qs, v_inds)
    logits = jnp.einsum('bnd,bnkd->bnk', ks, qs_g) / jnp.sqrt(jnp.float32(KQ))
    adj = jax.nn.softmax(logits, axis=-1)
    adj = adj / jnp.clip(jnp.max(adj, axis=-1, keepdims=True), 1e-12, None)
    batch_off = (jnp.arange(B, dtype=jnp.int32) * N)[:, None, None]
    u = jnp.broadcast_to(jnp.arange(N, dtype=jnp.int32)[None, :, None], (B, N, KTOT)) + batch_off
    v = v_inds.astype(jnp.int32) + batch_off
    u_f = u.reshape(-1)
    v_f = v.reshape(-1)
    val_f = adj.reshape(-1)
    h = h0.reshape(B * N, D)
    for _ in range(ITERS):
        msg = val_f[:, None] * h[u_f]
        h = jnp.zeros((B * N, D), dtype=jnp.float32).at[v_f].add(msg)
        h = jax.nn.relu(h)
        h = h / (jnp.linalg.norm(h, axis=-1, keepdims=True) + 1e-8)
    prop_map = h.reshape(B, W, H, D)
    return logits, prop_map


# speedup vs baseline: 31.3265x; 31.3265x over previous
import functools

import jax
import jax.numpy as jnp
from jax import lax
from jax.experimental import pallas as pl
from jax.experimental.pallas import tpu as pltpu
from jax.experimental.pallas import tpu_sc as plsc

H = 64; W = 64; B = 4; C = 128; KQ = 132; KLOC = 7; NLONG = 9; D = 64; ITERS = 16
N = H * W
KTOT = KLOC * KLOC + NLONG
KPAD = 64
R = B * N

NC = 2
NS = 16
NW = NC * NS
ROWS_PER_W = R // NW
CH = 4
NCH = ROWS_PER_W // CH
CBLK = 512


def _kq_body(x_ref, wk_ref, bk_ref, wq_ref, bq_ref, ks_ref, qs_ref):
    x = x_ref[0]
    scale = 1.0 / jnp.sqrt(jnp.float32(KQ))
    ks = jnp.dot(x, wk_ref[...], preferred_element_type=jnp.float32,
                 precision=lax.Precision.HIGHEST) + bk_ref[...]
    ks_ref[0] = ks * scale
    qs_ref[0] = jnp.dot(x, wq_ref[...], preferred_element_type=jnp.float32,
                        precision=lax.Precision.HIGHEST) + bq_ref[...]


def _compute_kq(x, Wk, bk, Wq, bq):
    return pl.pallas_call(
        _kq_body,
        grid=(B,),
        in_specs=[
            pl.BlockSpec((1, N, C), lambda b: (b, 0, 0)),
            pl.BlockSpec((C, KQ), lambda b: (0, 0)),
            pl.BlockSpec((KQ,), lambda b: (0,)),
            pl.BlockSpec((C, KQ), lambda b: (0, 0)),
            pl.BlockSpec((KQ,), lambda b: (0,)),
        ],
        out_specs=[
            pl.BlockSpec((1, N, KQ), lambda b: (b, 0, 0)),
            pl.BlockSpec((1, N, KQ), lambda b: (b, 0, 0)),
        ],
        out_shape=[
            jax.ShapeDtypeStruct((B, N, KQ), jnp.float32),
            jax.ShapeDtypeStruct((B, N, KQ), jnp.float32),
        ],
    )(x, Wk, bk, Wq, bq)


def _aff_body(ks_ref, qs_ref, l_ref):
    l_ref[0] = lax.dot_general(
        ks_ref[0], qs_ref[0], (((1,), (1,)), ((), ())),
        preferred_element_type=jnp.float32,
        precision=lax.Precision.HIGHEST)


def _compute_affinity(ks, qs):
    nrb = 8
    return pl.pallas_call(
        _aff_body,
        grid=(B, nrb),
        in_specs=[
            pl.BlockSpec((1, N // nrb, KQ), lambda b, r: (b, r, 0)),
            pl.BlockSpec((1, N, KQ), lambda b, r: (b, 0, 0)),
        ],
        out_specs=pl.BlockSpec((1, N // nrb, N), lambda b, r: (b, r, 0)),
        out_shape=jax.ShapeDtypeStruct((B, N, N), jnp.float32),
    )(ks, qs)


_MESH = plsc.VectorSubcoreMesh(
    core_axis_name="c", subcore_axis_name="s", num_cores=NC, num_subcores=NS)


@functools.partial(
    pl.kernel,
    out_type=(jax.ShapeDtypeStruct((R * KPAD,), jnp.float32),
              jax.ShapeDtypeStruct((R * N,), jnp.float32)),
    mesh=_MESH,
    scratch_types=[
        pltpu.VMEM((CH * N,), jnp.float32),
        pltpu.VMEM((CH * N,), jnp.float32),
        pltpu.VMEM((CH * KPAD,), jnp.int32),
        pltpu.VMEM((CH * KPAD,), jnp.int32),
        pltpu.VMEM((CH * N,), jnp.float32),
        pltpu.VMEM((CH * N,), jnp.float32),
        pltpu.VMEM((CH * KPAD,), jnp.float32),
        pltpu.VMEM((CH * KPAD,), jnp.float32),
        pltpu.SemaphoreType.DMA,
        pltpu.SemaphoreType.DMA,
        pltpu.SemaphoreType.DMA,
        pltpu.SemaphoreType.DMA,
        pltpu.SemaphoreType.DMA,
        pltpu.SemaphoreType.DMA,
        pltpu.SemaphoreType.DMA,
        pltpu.SemaphoreType.DMA,
    ],
    compiler_params=pltpu.CompilerParams(needs_layout_passes=False),
)
def _sc_adj(l_hbm, vi_hbm, lg_hbm, a_hbm,
            lb0, lb1, ib0, ib1, ab0, ab1, ob0, ob1,
            sl0, sl1, si0, si1, sa0, sa1, so0, so1):
    cid = lax.axis_index("c")
    sid = lax.axis_index("s")
    wid = sid * NC + cid
    base = wid * ROWS_PER_W
    lane = lax.iota(jnp.int32, 16)
    slots = ((lb0, ib0, ab0, ob0, sl0, si0, sa0, so0),
             (lb1, ib1, ab1, ob1, sl1, si1, sa1, so1))

    def start_in(c, lb, ib, sl, si):
        r0 = base + c * CH
        pltpu.async_copy(l_hbm.at[pl.ds(r0 * N, CH * N)], lb, sl)
        pltpu.async_copy(vi_hbm.at[pl.ds(r0 * KPAD, CH * KPAD)], ib, si)

    def zero_a(ab):
        zval = jnp.zeros((16,), jnp.float32)

        def zbody(i, _):
            plsc.store_scatter(ab, [lane + i * 16], zval)
            return 0
        lax.fori_loop(0, CH * N // 16, zbody, 0, unroll=8)

    def process_row(j, lb, ib, ab, ob):
        idxg = []
        xg = []
        for g in range(4):
            idx = ib[pl.ds(j * KPAD + g * 16, 16)]
            x = plsc.load_gather(lb, [idx + j * N])
            ob[pl.ds(j * KPAD + g * 16, 16)] = x
            idxg.append(idx)
            xg.append(x)
        tail = KTOT - 48
        x3m = jnp.where(lane < tail, xg[3], jnp.float32(-3e38))
        m = jnp.max(jnp.maximum(jnp.maximum(xg[0], xg[1]),
                                jnp.maximum(xg[2], x3m)))
        eg = [jnp.exp(x - m) for x in xg]
        eg[3] = jnp.where(lane < tail, eg[3], jnp.float32(0.0))
        for k in range(KTOT):
            g, l = divmod(k, 16)
            plsc.addupdate_scatter(ab, [idxg[g] + j * N], eg[g],
                                   mask=lane == l)

    def half(c, slot):
        lb, ib, ab, ob, sl, si, sa, so = slots[slot]
        r0 = base + c * CH

        @pl.when(c >= 2)
        def _():
            r0p = base + (c - 2) * CH
            pltpu.make_async_copy(ab, a_hbm.at[pl.ds(r0p * N, CH * N)],
                                  sa).wait()
            pltpu.make_async_copy(ob, lg_hbm.at[pl.ds(r0p * KPAD, CH * KPAD)],
                                  so).wait()

        zero_a(ab)
        pltpu.make_async_copy(l_hbm.at[pl.ds(r0 * N, CH * N)], lb, sl).wait()
        pltpu.make_async_copy(vi_hbm.at[pl.ds(r0 * KPAD, CH * KPAD)], ib,
                              si).wait()
        for j in range(CH):
            process_row(j, lb, ib, ab, ob)
        pltpu.async_copy(ab, a_hbm.at[pl.ds(r0 * N, CH * N)], sa)
        pltpu.async_copy(ob, lg_hbm.at[pl.ds(r0 * KPAD, CH * KPAD)], so)

        @pl.when(c + 2 < NCH)
        def _():
            start_in(c + 2, lb, ib, sl, si)

    start_in(0, lb0, ib0, sl0, si0)
    start_in(1, lb1, ib1, sl1, si1)

    def main_body(i, _):
        half(2 * i, 0)
        half(2 * i + 1, 1)
        return 0

    lax.fori_loop(0, NCH // 2, main_body, 0)

    for c, slot in ((NCH - 2, 0), (NCH - 1, 1)):
        lb, ib, ab, ob, sl, si, sa, so = slots[slot]
        r0 = base + c * CH
        pltpu.make_async_copy(ab, a_hbm.at[pl.ds(r0 * N, CH * N)], sa).wait()
        pltpu.make_async_copy(ob, lg_hbm.at[pl.ds(r0 * KPAD, CH * KPAD)],
                              so).wait()


def _prop_body(h0_ref, a_ref, out_ref, hbuf):
    t = pl.program_id(0)
    b = pl.program_id(1)
    c = pl.program_id(2)

    @pl.when(t == 0)
    def _():
        hbuf[0, b, :, pl.ds(c * CBLK, CBLK)] = h0_ref[0]
        out_ref[0] = h0_ref[0]

    @pl.when(t > 0)
    def _():
        src = hbuf[(t + 1) % 2, b]
        y = lax.dot_general(src, a_ref[0], (((1,), (0,)), ((), ())),
                            preferred_element_type=jnp.float32,
                            precision=lax.Precision.HIGHEST)
        y = jnp.maximum(y, 0.0)
        nrm = jnp.sqrt(jnp.sum(y * y, axis=0, keepdims=True))
        y = y / (nrm + 1e-8)
        hbuf[t % 2, b, :, pl.ds(c * CBLK, CBLK)] = y
        out_ref[0] = y


def _propagate(h0t, a):
    ncb = N // CBLK
    return pl.pallas_call(
        _prop_body,
        grid=(ITERS + 1, B, ncb),
        in_specs=[
            pl.BlockSpec((1, D, CBLK), lambda t, b, c: (b, 0, c)),
            pl.BlockSpec((1, N, CBLK), lambda t, b, c: (b, 0, c)),
        ],
        out_specs=pl.BlockSpec((1, D, CBLK), lambda t, b, c: (b, 0, c)),
        out_shape=jax.ShapeDtypeStruct((B, D, N), jnp.float32),
        scratch_shapes=[pltpu.VMEM((2, B, D, N), jnp.float32)],
    )(h0t, a)


def kernel(x, v_inds, h0, Wk, bk, Wq, bq):
    ks, qs = _compute_kq(x, Wk, bk, Wq, bq)
    l_full = _compute_affinity(ks, qs)
    vip = jnp.pad(v_inds.reshape(R, KTOT).astype(jnp.int32),
                  ((0, 0), (0, KPAD - KTOT)))
    logits_pad, a = _sc_adj(l_full.reshape(R * N), vip.reshape(R * KPAD))
    logits = logits_pad.reshape(R, KPAD)[:, :KTOT].reshape(B, N, KTOT)
    h0t = jnp.swapaxes(h0, 1, 2)
    ht = _propagate(h0t, a.reshape(B, N, N))
    prop_map = jnp.swapaxes(ht, 1, 2).reshape(B, W, H, D)
    return logits, prop_map


# speedup vs baseline: 61.1946x; 1.9534x over previous
import functools

import jax
import jax.numpy as jnp
from jax import lax
from jax.experimental import pallas as pl
from jax.experimental.pallas import tpu as pltpu
from jax.experimental.pallas import tpu_sc as plsc

H = 64; W = 64; B = 4; C = 128; KQ = 132; KLOC = 7; NLONG = 9; D = 64; ITERS = 16
N = H * W
KTOT = KLOC * KLOC + NLONG
KPAD = 64
R = B * N

NC = 2
NS = 16
NW = NC * NS
ROWS_PER_W = R // NW
CH = 4
NCH = ROWS_PER_W // CH
CBLK = 512


def _kq_body(x_ref, wk_ref, bk_ref, wq_ref, bq_ref, ks_ref, qs_ref):
    x = x_ref[0]
    scale = 1.0 / jnp.sqrt(jnp.float32(KQ))
    ks = jnp.dot(x, wk_ref[...], preferred_element_type=jnp.float32,
                 precision=lax.Precision.HIGHEST) + bk_ref[...]
    ks_ref[0] = ks * scale
    qs_ref[0] = jnp.dot(x, wq_ref[...], preferred_element_type=jnp.float32,
                        precision=lax.Precision.HIGHEST) + bq_ref[...]


def _compute_kq(x, Wk, bk, Wq, bq):
    return pl.pallas_call(
        _kq_body,
        grid=(B,),
        in_specs=[
            pl.BlockSpec((1, N, C), lambda b: (b, 0, 0)),
            pl.BlockSpec((C, KQ), lambda b: (0, 0)),
            pl.BlockSpec((KQ,), lambda b: (0,)),
            pl.BlockSpec((C, KQ), lambda b: (0, 0)),
            pl.BlockSpec((KQ,), lambda b: (0,)),
        ],
        out_specs=[
            pl.BlockSpec((1, N, KQ), lambda b: (b, 0, 0)),
            pl.BlockSpec((1, N, KQ), lambda b: (b, 0, 0)),
        ],
        out_shape=[
            jax.ShapeDtypeStruct((B, N, KQ), jnp.float32),
            jax.ShapeDtypeStruct((B, N, KQ), jnp.float32),
        ],
    )(x, Wk, bk, Wq, bq)


def _aff_body(ks_ref, qs_ref, l_ref):
    l_ref[0] = lax.dot_general(
        ks_ref[0], qs_ref[0], (((1,), (1,)), ((), ())),
        preferred_element_type=jnp.float32,
        precision=lax.Precision.HIGHEST)


def _compute_affinity(ks, qs):
    nrb = 8
    return pl.pallas_call(
        _aff_body,
        grid=(B, nrb),
        in_specs=[
            pl.BlockSpec((1, N // nrb, KQ), lambda b, r: (b, r, 0)),
            pl.BlockSpec((1, N, KQ), lambda b, r: (b, 0, 0)),
        ],
        out_specs=pl.BlockSpec((1, N // nrb, N), lambda b, r: (b, r, 0)),
        out_shape=jax.ShapeDtypeStruct((B, N, N), jnp.float32),
    )(ks, qs)


_MESH = plsc.VectorSubcoreMesh(
    core_axis_name="c", subcore_axis_name="s", num_cores=NC, num_subcores=NS)


@functools.partial(
    pl.kernel,
    out_type=(jax.ShapeDtypeStruct((R * KPAD,), jnp.float32),
              jax.ShapeDtypeStruct((B, N, N // 2), jnp.int32)),
    mesh=_MESH,
    scratch_types=[
        pltpu.VMEM((CH * N,), jnp.float32),
        pltpu.VMEM((CH * N,), jnp.float32),
        pltpu.VMEM((CH * KPAD,), jnp.int32),
        pltpu.VMEM((CH * KPAD,), jnp.int32),
        pltpu.VMEM((CH, N // 2), jnp.int32),
        pltpu.VMEM((CH, N // 2), jnp.int32),
        pltpu.VMEM((CH * KPAD,), jnp.float32),
        pltpu.VMEM((CH * KPAD,), jnp.float32),
        pltpu.VMEM((N,), jnp.float32),
        pltpu.SemaphoreType.DMA,
        pltpu.SemaphoreType.DMA,
        pltpu.SemaphoreType.DMA,
        pltpu.SemaphoreType.DMA,
        pltpu.SemaphoreType.DMA,
        pltpu.SemaphoreType.DMA,
        pltpu.SemaphoreType.DMA,
        pltpu.SemaphoreType.DMA,
    ],
    compiler_params=pltpu.CompilerParams(needs_layout_passes=False),
)
def _sc_adj(l_hbm, vi_hbm, lg_hbm, a_hbm,
            lb0, lb1, ib0, ib1, wb0, wb1, ob0, ob1, acc,
            sl0, sl1, si0, si1, sa0, sa1, so0, so1):
    cid = lax.axis_index("c")
    sid = lax.axis_index("s")
    wid = sid * NC + cid
    base = wid * ROWS_PER_W
    lane = lax.iota(jnp.int32, 16)
    zval = jnp.zeros((16,), jnp.float32)
    zw = jnp.zeros((16,), jnp.int32)
    slots = ((lb0, ib0, wb0, ob0, sl0, si0, sa0, so0),
             (lb1, ib1, wb1, ob1, sl1, si1, sa1, so1))

    def start_in(c, lb, ib, sl, si):
        r0 = base + c * CH
        pltpu.async_copy(l_hbm.at[pl.ds(r0 * N, CH * N)], lb, sl)
        pltpu.async_copy(vi_hbm.at[pl.ds(r0 * KPAD, CH * KPAD)], ib, si)

    def zbody0(i, _):
        plsc.store_scatter(acc, [lane + i * 16], zval)
        return 0
    lax.fori_loop(0, N // 16, zbody0, 0, unroll=8)

    def zero_words(wb):
        def zbody(i, _):
            pos = lane + i * 16
            for j in range(CH):
                plsc.store_scatter(wb, [jnp.full((16,), j, jnp.int32), pos],
                                   zw)
            return 0
        lax.fori_loop(0, N // 2 // 16, zbody, 0, unroll=8)

    def process_row(j, lb, ib, wb, ob):
        idxg = []
        xg = []
        for g in range(4):
            idx = ib[pl.ds(j * KPAD + g * 16, 16)]
            x = plsc.load_gather(lb, [idx + j * N])
            ob[pl.ds(j * KPAD + g * 16, 16)] = x
            idxg.append(idx)
            xg.append(x)
        tail = KTOT - 48
        x3m = jnp.where(lane < tail, xg[3], jnp.float32(-3e38))
        m = jnp.max(jnp.maximum(jnp.maximum(xg[0], xg[1]),
                                jnp.maximum(xg[2], x3m)))
        eg = [jnp.exp(x - m) for x in xg]
        eg[3] = jnp.where(lane < tail, eg[3], jnp.float32(0.0))
        for k in range(KTOT):
            g, l = divmod(k, 16)
            plsc.addupdate_scatter(acc, [idxg[g]], eg[g], mask=lane == l)
        for g in range(4):
            evenbase = (idxg[g] >> 1) << 1
            plo = plsc.load_gather(acc, [evenbase])
            phi = plsc.load_gather(acc, [evenbase + 1])
            w = plsc.bitcast(
                plsc.pack(plo, phi, format=plsc.PackFormat.INTERLEAVED),
                jnp.int32)
            plsc.store_scatter(wb, [jnp.full((16,), j, jnp.int32),
                                    idxg[g] >> 1], w)
        for g in range(4):
            plsc.store_scatter(acc, [idxg[g]], zval)

    def half(c, slot):
        lb, ib, wb, ob, sl, si, sa, so = slots[slot]
        r0 = base + c * CH

        @pl.when(c >= 2)
        def _():
            r0p = base + (c - 2) * CH
            bb = r0p // N
            pltpu.make_async_copy(wb, a_hbm.at[bb, pl.ds(r0p % N, CH)],
                                  sa).wait()
            pltpu.make_async_copy(ob, lg_hbm.at[pl.ds(r0p * KPAD, CH * KPAD)],
                                  so).wait()

        zero_words(wb)
        pltpu.make_async_copy(l_hbm.at[pl.ds(r0 * N, CH * N)], lb, sl).wait()
        pltpu.make_async_copy(vi_hbm.at[pl.ds(r0 * KPAD, CH * KPAD)], ib,
                              si).wait()
        for j in range(CH):
            process_row(j, lb, ib, wb, ob)
        pltpu.async_copy(wb, a_hbm.at[r0 // N, pl.ds(r0 % N, CH)], sa)
        pltpu.async_copy(ob, lg_hbm.at[pl.ds(r0 * KPAD, CH * KPAD)], so)

        @pl.when(c + 2 < NCH)
        def _():
            start_in(c + 2, lb, ib, sl, si)

    start_in(0, lb0, ib0, sl0, si0)
    start_in(1, lb1, ib1, sl1, si1)

    def main_body(i, _):
        half(2 * i, 0)
        half(2 * i + 1, 1)
        return 0

    lax.fori_loop(0, NCH // 2, main_body, 0)

    for c, slot in ((NCH - 2, 0), (NCH - 1, 1)):
        lb, ib, wb, ob, sl, si, sa, so = slots[slot]
        r0 = base + c * CH
        pltpu.make_async_copy(wb, a_hbm.at[r0 // N, pl.ds(r0 % N, CH)],
                              sa).wait()
        pltpu.make_async_copy(ob, lg_hbm.at[pl.ds(r0 * KPAD, CH * KPAD)],
                              so).wait()


def _prop_body(h0_ref, abits_ref, out_ref, avm, hbuf, sem):
    b = pl.program_id(0)
    t = pl.program_id(1)

    @pl.when(t == 0)
    def _():
        cp = pltpu.make_async_copy(abits_ref.at[b], avm, sem)
        cp.start()
        hbuf[0] = h0_ref[0]
        cp.wait()

    @pl.when(t > 0)
    def _():
        srcb = hbuf[(t + 1) % 2].astype(jnp.bfloat16)
        for c in range(N // CBLK):
            blk = avm[:, pl.ds(c * CBLK, CBLK)]
            y = lax.dot_general(srcb, blk, (((1,), (0,)), ((), ())),
                                preferred_element_type=jnp.float32)
            y = jnp.maximum(y, 0.0)
            nrm = jnp.sqrt(jnp.sum(y * y, axis=0, keepdims=True))
            y = y / (nrm + 1e-8)
            hbuf[t % 2, :, pl.ds(c * CBLK, CBLK)] = y

            @pl.when(t == ITERS)
            def _():
                out_ref[0, :, pl.ds(c * CBLK, CBLK)] = y


def _propagate(h0t, a_bits):
    return pl.pallas_call(
        _prop_body,
        grid=(B, ITERS + 1),
        in_specs=[
            pl.BlockSpec((1, D, N), lambda b, t: (b, 0, 0)),
            pl.BlockSpec(memory_space=pl.ANY),
        ],
        out_specs=pl.BlockSpec((1, D, N), lambda b, t: (b, 0, 0)),
        out_shape=jax.ShapeDtypeStruct((B, D, N), jnp.float32),
        scratch_shapes=[
            pltpu.VMEM((N, N), jnp.bfloat16),
            pltpu.VMEM((2, D, N), jnp.float32),
            pltpu.SemaphoreType.DMA,
        ],
    )(h0t, a_bits)


def kernel(x, v_inds, h0, Wk, bk, Wq, bq):
    ks, qs = _compute_kq(x, Wk, bk, Wq, bq)
    l_full = _compute_affinity(ks, qs)
    vip = jnp.pad(v_inds.reshape(R, KTOT).astype(jnp.int32),
                  ((0, 0), (0, KPAD - KTOT)))
    logits_pad, a_bits = _sc_adj(l_full.reshape(R * N), vip.reshape(R * KPAD))
    logits = logits_pad.reshape(R, KPAD)[:, :KTOT].reshape(B, N, KTOT)
    abf = lax.bitcast_convert_type(a_bits, jnp.bfloat16).reshape(B, N, N)
    h0t = jnp.swapaxes(h0, 1, 2)
    ht = _propagate(h0t, abf)
    prop_map = jnp.swapaxes(ht, 1, 2).reshape(B, W, H, D)
    return logits, prop_map


# speedup vs baseline: 66.4738x; 1.0863x over previous
import functools

import jax
import jax.numpy as jnp
from jax import lax
from jax.experimental import pallas as pl
from jax.experimental.pallas import tpu as pltpu
from jax.experimental.pallas import tpu_sc as plsc

H = 64; W = 64; B = 4; C = 128; KQ = 132; KLOC = 7; NLONG = 9; D = 64; ITERS = 16
N = H * W
KTOT = KLOC * KLOC + NLONG
KPAD = 64
R = B * N

NC = 2
NS = 16
NW = NC * NS
ROWS_PER_W = R // NW
CH = 4
NCH = ROWS_PER_W // CH
CBLK = 512


def _kq_body(x_ref, wk_ref, bk_ref, wq_ref, bq_ref, ks_ref, qs_ref):
    x = x_ref[0]
    scale = 1.0 / jnp.sqrt(jnp.float32(KQ))
    ks = jnp.dot(x, wk_ref[...], preferred_element_type=jnp.float32,
                 precision=lax.Precision.HIGHEST) + bk_ref[...]
    ks_ref[0] = ks * scale
    qs_ref[0] = jnp.dot(x, wq_ref[...], preferred_element_type=jnp.float32,
                        precision=lax.Precision.HIGHEST) + bq_ref[...]


def _compute_kq(x, Wk, bk, Wq, bq):
    return pl.pallas_call(
        _kq_body,
        grid=(B,),
        in_specs=[
            pl.BlockSpec((1, N, C), lambda b: (b, 0, 0)),
            pl.BlockSpec((C, KQ), lambda b: (0, 0)),
            pl.BlockSpec((KQ,), lambda b: (0,)),
            pl.BlockSpec((C, KQ), lambda b: (0, 0)),
            pl.BlockSpec((KQ,), lambda b: (0,)),
        ],
        out_specs=[
            pl.BlockSpec((1, N, KQ), lambda b: (b, 0, 0)),
            pl.BlockSpec((1, N, KQ), lambda b: (b, 0, 0)),
        ],
        out_shape=[
            jax.ShapeDtypeStruct((B, N, KQ), jnp.float32),
            jax.ShapeDtypeStruct((B, N, KQ), jnp.float32),
        ],
    )(x, Wk, bk, Wq, bq)


def _aff_body(ks_ref, qs_ref, l_ref):
    y = lax.dot_general(
        ks_ref[0], qs_ref[0], (((1,), (1,)), ((), ())),
        preferred_element_type=jnp.float32,
        precision=lax.Precision.HIGHEST)
    l_ref[...] = y.reshape(N // 8 * N)


def _compute_affinity(ks, qs):
    nrb = 8
    return pl.pallas_call(
        _aff_body,
        grid=(B, nrb),
        in_specs=[
            pl.BlockSpec((1, N // nrb, KQ), lambda b, r: (b, r, 0)),
            pl.BlockSpec((1, N, KQ), lambda b, r: (b, 0, 0)),
        ],
        out_specs=pl.BlockSpec((N // nrb * N,), lambda b, r: (b * nrb + r,)),
        out_shape=jax.ShapeDtypeStruct((R * N,), jnp.float32),
    )(ks, qs)


_MESH = plsc.VectorSubcoreMesh(
    core_axis_name="c", subcore_axis_name="s", num_cores=NC, num_subcores=NS)


@functools.partial(
    pl.kernel,
    out_type=(jax.ShapeDtypeStruct((R * KPAD,), jnp.float32),
              jax.ShapeDtypeStruct((B, N, N // 2), jnp.int32)),
    mesh=_MESH,
    scratch_types=[
        pltpu.VMEM((CH * N,), jnp.float32),
        pltpu.VMEM((CH * N,), jnp.float32),
        pltpu.VMEM((CH * KPAD,), jnp.int32),
        pltpu.VMEM((CH * KPAD,), jnp.int32),
        pltpu.VMEM((CH, N // 2), jnp.int32),
        pltpu.VMEM((CH, N // 2), jnp.int32),
        pltpu.VMEM((CH * KPAD,), jnp.float32),
        pltpu.VMEM((CH * KPAD,), jnp.float32),
        pltpu.VMEM((N,), jnp.float32),
        pltpu.SemaphoreType.DMA,
        pltpu.SemaphoreType.DMA,
        pltpu.SemaphoreType.DMA,
        pltpu.SemaphoreType.DMA,
        pltpu.SemaphoreType.DMA,
        pltpu.SemaphoreType.DMA,
        pltpu.SemaphoreType.DMA,
        pltpu.SemaphoreType.DMA,
    ],
    compiler_params=pltpu.CompilerParams(needs_layout_passes=False),
)
def _sc_adj(l_hbm, vi_hbm, lg_hbm, a_hbm,
            lb0, lb1, ib0, ib1, wb0, wb1, ob0, ob1, acc,
            sl0, sl1, si0, si1, sa0, sa1, so0, so1):
    cid = lax.axis_index("c")
    sid = lax.axis_index("s")
    wid = sid * NC + cid
    base = wid * ROWS_PER_W
    lane = lax.iota(jnp.int32, 16)
    zval = jnp.zeros((16,), jnp.float32)
    zw = jnp.zeros((16,), jnp.int32)
    slots = ((lb0, ib0, wb0, ob0, sl0, si0, sa0, so0),
             (lb1, ib1, wb1, ob1, sl1, si1, sa1, so1))

    def start_in(c, lb, ib, sl, si):
        r0 = base + c * CH
        pltpu.async_copy(l_hbm.at[pl.ds(r0 * N, CH * N)], lb, sl)
        pltpu.async_copy(vi_hbm.at[pl.ds(r0 * KPAD, CH * KPAD)], ib, si)

    def zbody0(i, _):
        plsc.store_scatter(acc, [lane + i * 16], zval)
        return 0
    lax.fori_loop(0, N // 16, zbody0, 0, unroll=8)

    def zero_words(wb):
        def zbody(i, _):
            pos = lane + i * 16
            for j in range(CH):
                plsc.store_scatter(wb, [jnp.full((16,), j, jnp.int32), pos],
                                   zw)
            return 0
        lax.fori_loop(0, N // 2 // 16, zbody, 0, unroll=8)

    def process_row(j, lb, ib, wb, ob):
        idxg = []
        xg = []
        for g in range(4):
            idx = ib[pl.ds(j * KPAD + g * 16, 16)]
            x = plsc.load_gather(lb, [idx + j * N])
            ob[pl.ds(j * KPAD + g * 16, 16)] = x
            idxg.append(idx)
            xg.append(x)
        tail = KTOT - 48
        x3m = jnp.where(lane < tail, xg[3], jnp.float32(-3e38))
        m = jnp.max(jnp.maximum(jnp.maximum(xg[0], xg[1]),
                                jnp.maximum(xg[2], x3m)))
        eg = [jnp.exp(x - m) for x in xg]
        eg[3] = jnp.where(lane < tail, eg[3], jnp.float32(0.0))
        for k in range(KTOT):
            g, l = divmod(k, 16)
            plsc.addupdate_scatter(acc, [idxg[g]], eg[g], mask=lane == l)
        for g in range(4):
            evenbase = (idxg[g] >> 1) << 1
            plo = plsc.load_gather(acc, [evenbase])
            phi = plsc.load_gather(acc, [evenbase + 1])
            w = plsc.bitcast(
                plsc.pack(plo, phi, format=plsc.PackFormat.INTERLEAVED),
                jnp.int32)
            plsc.store_scatter(wb, [jnp.full((16,), j, jnp.int32),
                                    idxg[g] >> 1], w)
        for g in range(4):
            plsc.store_scatter(acc, [idxg[g]], zval)

    def half(c, slot):
        lb, ib, wb, ob, sl, si, sa, so = slots[slot]
        r0 = base + c * CH

        @pl.when(c >= 2)
        def _():
            r0p = base + (c - 2) * CH
            bb = r0p // N
            pltpu.make_async_copy(wb, a_hbm.at[bb, pl.ds(r0p % N, CH)],
                                  sa).wait()
            pltpu.make_async_copy(ob, lg_hbm.at[pl.ds(r0p * KPAD, CH * KPAD)],
                                  so).wait()

        zero_words(wb)
        pltpu.make_async_copy(l_hbm.at[pl.ds(r0 * N, CH * N)], lb, sl).wait()
        pltpu.make_async_copy(vi_hbm.at[pl.ds(r0 * KPAD, CH * KPAD)], ib,
                              si).wait()
        for j in range(CH):
            process_row(j, lb, ib, wb, ob)
        pltpu.async_copy(wb, a_hbm.at[r0 // N, pl.ds(r0 % N, CH)], sa)
        pltpu.async_copy(ob, lg_hbm.at[pl.ds(r0 * KPAD, CH * KPAD)], so)

        @pl.when(c + 2 < NCH)
        def _():
            start_in(c + 2, lb, ib, sl, si)

    start_in(0, lb0, ib0, sl0, si0)
    start_in(1, lb1, ib1, sl1, si1)

    def main_body(i, _):
        half(2 * i, 0)
        half(2 * i + 1, 1)
        return 0

    lax.fori_loop(0, NCH // 2, main_body, 0)

    for c, slot in ((NCH - 2, 0), (NCH - 1, 1)):
        lb, ib, wb, ob, sl, si, sa, so = slots[slot]
        r0 = base + c * CH
        pltpu.make_async_copy(wb, a_hbm.at[r0 // N, pl.ds(r0 % N, CH)],
                              sa).wait()
        pltpu.make_async_copy(ob, lg_hbm.at[pl.ds(r0 * KPAD, CH * KPAD)],
                              so).wait()


def _prop_body(h0_ref, abits_ref, out_ref, avm, hbuf, sem):
    b = pl.program_id(0)
    t = pl.program_id(1)

    @pl.when(t == 0)
    def _():
        cp = pltpu.make_async_copy(abits_ref.at[b], avm, sem)
        cp.start()
        hbuf[0] = h0_ref[0]
        cp.wait()

    @pl.when(t > 0)
    def _():
        srcb = hbuf[(t + 1) % 2].astype(jnp.bfloat16)
        for c in range(N // CBLK):
            blk = avm[:, pl.ds(c * CBLK, CBLK)]
            y = lax.dot_general(srcb, blk, (((1,), (0,)), ((), ())),
                                preferred_element_type=jnp.float32)
            y = jnp.maximum(y, 0.0)
            nrm = jnp.sqrt(jnp.sum(y * y, axis=0, keepdims=True))
            y = y / (nrm + 1e-8)
            hbuf[t % 2, :, pl.ds(c * CBLK, CBLK)] = y

            @pl.when(t == ITERS)
            def _():
                out_ref[0, :, pl.ds(c * CBLK, CBLK)] = y


def _propagate(h0t, a_bits):
    return pl.pallas_call(
        _prop_body,
        grid=(B, ITERS + 1),
        in_specs=[
            pl.BlockSpec((1, D, N), lambda b, t: (b, 0, 0)),
            pl.BlockSpec(memory_space=pl.ANY),
        ],
        out_specs=pl.BlockSpec((1, D, N), lambda b, t: (b, 0, 0)),
        out_shape=jax.ShapeDtypeStruct((B, D, N), jnp.float32),
        scratch_shapes=[
            pltpu.VMEM((N, N), jnp.bfloat16),
            pltpu.VMEM((2, D, N), jnp.float32),
            pltpu.SemaphoreType.DMA,
        ],
    )(h0t, a_bits)


def kernel(x, v_inds, h0, Wk, bk, Wq, bq):
    ks, qs = _compute_kq(x, Wk, bk, Wq, bq)
    l_full = _compute_affinity(ks, qs)
    vip = jnp.pad(v_inds.reshape(R, KTOT).astype(jnp.int32),
                  ((0, 0), (0, KPAD - KTOT)))
    logits_pad, a_bits = _sc_adj(l_full, vip.reshape(R * KPAD))
    logits = logits_pad.reshape(R, KPAD)[:, :KTOT].reshape(B, N, KTOT)
    abf = lax.bitcast_convert_type(a_bits, jnp.bfloat16).reshape(B, N, N)
    h0t = jnp.swapaxes(h0, 1, 2)
    ht = _propagate(h0t, abf)
    prop_map = jnp.swapaxes(ht, 1, 2).reshape(B, W, H, D)
    return logits, prop_map
